# Initial kernel scaffold; baseline (speedup 1.0000x reference)
import jax
import jax.numpy as jnp
from jax.experimental import pallas as pl


def kernel(pharm_scalars, pharm_pos, pharm_vectors, prot_scalars, prot_pos, prot_vectors, params, edge_index_ff, edge_index_pf, edge_index_fp, edge_index_pp, pharm_batch_idx, prot_batch_idx):
    raise NotImplementedError("write your pallas kernel here")


# speedup vs baseline: 3.6340x; 3.6340x over previous
import functools

import jax
import jax.numpy as jnp
from jax.experimental import pallas as pl
from jax.experimental.pallas import tpu as pltpu

S = 128
VD = 16
E = 160000
MSG_NORM = 10.0
INTER = 64
OUT = 128
ETYPES = [('ff', 'pharm', 'pharm'), ('pf', 'prot', 'pharm'),
          ('fp', 'pharm', 'prot'), ('pp', 'prot', 'prot')]

EB = 1280
NB = 1000

_F32 = jnp.float32


def _dot(a, b):
    return jax.lax.dot_general(a, b, (((1,), (0,)), ((), ())),
                               preferred_element_type=_F32)


def _gvp16(s, vx, vy, vz, wht, wut, wsst, wsht, bs):
    vhx = _dot(vx, wht)
    vhy = _dot(vy, wht)
    vhz = _dot(vz, wht)
    sh = jnp.sqrt(vhx * vhx + vhy * vhy + vhz * vhz + 1e-8)
    so = jnp.maximum(_dot(s, wsst) + _dot(sh, wsht) + bs, 0.0)
    vux = _dot(vhx, wut)
    vuy = _dot(vhy, wut)
    vuz = _dot(vhz, wut)
    n = jnp.sqrt(vux * vux + vuy * vuy + vuz * vuz + 1e-8)
    g = jax.nn.sigmoid(n)
    return so, vux * g, vuy * g, vuz * g


def _msg_body(sg, vg, ps, pd,
              a0t, w0l, wu0t, ws0st, ws0d, ws0ht, bs0,
              wh1t, wu1t, ws1st, ws1ht, bs1,
              wh2t, wu2t, ws2st, ws2ht, bs2,
              sm_o, vm_o):
    s = sg[...]
    vx = vg[:, 0:VD]
    vy = vg[:, VD:2 * VD]
    vz = vg[:, 2 * VD:3 * VD]
    rx = ps[:, 0:1] - pd[:, 0:1]
    ry = ps[:, 1:2] - pd[:, 1:2]
    rz = ps[:, 2:3] - pd[:, 2:3]
    dist = jnp.sqrt(rx * rx + ry * ry + rz * rz + 1e-8)

    vhx = _dot(vx, a0t[0]) + rx * w0l[0]
    vhy = _dot(vy, a0t[0]) + ry * w0l[0]
    vhz = _dot(vz, a0t[0]) + rz * w0l[0]
    sh = jnp.sqrt(vhx * vhx + vhy * vhy + vhz * vhz + 1e-8)
    so = jnp.maximum(_dot(s, ws0st[0]) + dist * ws0d[0] + _dot(sh, ws0ht[0])
                     + bs0[0], 0.0)
    vux = _dot(vhx, wu0t[0])
    vuy = _dot(vhy, wu0t[0])
    vuz = _dot(vhz, wu0t[0])
    n = jnp.sqrt(vux * vux + vuy * vuy + vuz * vuz + 1e-8)
    g = jax.nn.sigmoid(n)
    s, vx, vy, vz = so, vux * g, vuy * g, vuz * g

    s, vx, vy, vz = _gvp16(s, vx, vy, vz, wh1t[0], wu1t[0], ws1st[0],
                           ws1ht[0], bs1[0])
    s, vx, vy, vz = _gvp16(s, vx, vy, vz, wh2t[0], wu2t[0], ws2st[0],
                           ws2ht[0], bs2[0])

    inv = _F32(1.0 / MSG_NORM)
    sm_o[...] = s * inv
    vm_o[:, 0:VD] = vx * inv
    vm_o[:, VD:2 * VD] = vy * inv
    vm_o[:, 2 * VD:3 * VD] = vz * inv


def _msg_weights(layer):
    per_type = []
    for et, _st, _dt in ETYPES:
        g0, g1, g2 = layer['msg'][et]
        w = [
            g0['Wh'][:, :VD].T,
            g0['Wh'][:, VD:].T,
            g0['Wu'].T,
            g0['Ws'][:, :S].T,
            g0['Ws'][:, S:S + 1].T,
            g0['Ws'][:, S + 1:].T,
            g0['bs'][None, :],
        ]
        for g in (g1, g2):
            w += [
                g['Wh'].T,
                g['Wu'].T,
                g['Ws'][:, :S].T,
                g['Ws'][:, S:].T,
                g['bs'][None, :],
            ]
        per_type.append(w)
    return [jnp.stack([pt[k] for pt in per_type])
            for k in range(len(per_type[0]))]


def _run_msg(sg, vg, ps, pd, wts):
    ne = sg.shape[0]
    grid = (ne // EB,)
    bpt = E // EB
    espec = lambda w: pl.BlockSpec((EB, w), lambda i: (i, 0))
    wspec = lambda a: pl.BlockSpec((1,) + a.shape[1:],
                                   lambda i: (i // bpt, 0, 0))
    return pl.pallas_call(
        _msg_body,
        grid=grid,
        in_specs=[espec(S), espec(3 * VD), espec(8), espec(8)]
                 + [wspec(a) for a in wts],
        out_specs=[espec(S), espec(3 * VD)],
        out_shape=[jax.ShapeDtypeStruct((ne, S), _F32),
                   jax.ShapeDtypeStruct((ne, 3 * VD), _F32)],
    )(sg, vg, ps, pd, *wts)


def _upd_body(s0, v0, ags, agv,
              a0t, b0t, wu0t, wss0t, wsa0t, wsh0t, bs0,
              wh1t, wu1t, ws1st, ws1ht, bs1,
              s_o, v_o):
    vx0 = v0[:, 0:VD]
    vy0 = v0[:, VD:2 * VD]
    vz0 = v0[:, 2 * VD:3 * VD]
    ax = agv[:, 0:VD]
    ay = agv[:, VD:2 * VD]
    az = agv[:, 2 * VD:3 * VD]
    vhx = _dot(vx0, a0t[0]) + _dot(ax, b0t[0])
    vhy = _dot(vy0, a0t[0]) + _dot(ay, b0t[0])
    vhz = _dot(vz0, a0t[0]) + _dot(az, b0t[0])
    sh = jnp.sqrt(vhx * vhx + vhy * vhy + vhz * vhz + 1e-8)
    so = jnp.maximum(_dot(s0[...], wss0t[0]) + _dot(ags[...], wsa0t[0])
                     + _dot(sh, wsh0t[0]) + bs0[0], 0.0)
    vux = _dot(vhx, wu0t[0])
    vuy = _dot(vhy, wu0t[0])
    vuz = _dot(vhz, wu0t[0])
    n = jnp.sqrt(vux * vux + vuy * vuy + vuz * vuz + 1e-8)
    g = jax.nn.sigmoid(n)
    s, vx, vy, vz = so, vux * g, vuy * g, vuz * g

    s, vx, vy, vz = _gvp16(s, vx, vy, vz, wh1t[0], wu1t[0], ws1st[0],
                           ws1ht[0], bs1[0])
    s_o[...] = s0[...] + s
    v_o[:, 0:VD] = vx0 + vx
    v_o[:, VD:2 * VD] = vy0 + vy
    v_o[:, 2 * VD:3 * VD] = vz0 + vz


def _upd_weights(layer):
    per_type = []
    for t in ('pharm', 'prot'):
        g0, g1 = layer['upd'][t]
        w = [
            g0['Wh'][:, :VD].T,
            g0['Wh'][:, VD:].T,
            g0['Wu'].T,
            g0['Ws'][:, :S].T,
            g0['Ws'][:, S:2 * S].T,
            g0['Ws'][:, 2 * S:].T,
            g0['bs'][None, :],
            g1['Wh'].T,
            g1['Wu'].T,
            g1['Ws'][:, :S].T,
            g1['Ws'][:, S:].T,
            g1['bs'][None, :],
        ]
        per_type.append(w)
    return [jnp.stack([pt[k] for pt in per_type])
            for k in range(len(per_type[0]))]


def _run_upd(s0, v0, ags, agv, wts, bpt):
    nn = s0.shape[0]
    grid = (nn // NB,)
    nspec = lambda w: pl.BlockSpec((NB, w), lambda i: (i, 0))
    wspec = lambda a: pl.BlockSpec((1,) + a.shape[1:],
                                   lambda i: (i // bpt, 0, 0))
    return pl.pallas_call(
        _upd_body,
        grid=grid,
        in_specs=[nspec(S), nspec(3 * VD), nspec(S), nspec(3 * VD)]
                 + [wspec(a) for a in wts],
        out_specs=[nspec(S), nspec(3 * VD)],
        out_shape=[jax.ShapeDtypeStruct((nn, S), _F32),
                   jax.ShapeDtypeStruct((nn, 3 * VD), _F32)],
    )(s0, v0, ags, agv, *wts)


def _noise_body(s_in, v_in,
                wh0t, wu0t, ws0st, ws0ht, bs0,
                wh1t, wu1t, ws1st, ws1ht, bs1,
                wh2t, wu2t, ws2st, ws2ht, bs2,
                wt, b,
                s_o, v_o):
    s = s_in[...]
    vx = v_in[:, 0:VD]
    vy = v_in[:, VD:2 * VD]
    vz = v_in[:, 2 * VD:3 * VD]
    s, vx, vy, vz = _gvp16(s, vx, vy, vz, wh0t[...], wu0t[...], ws0st[...],
                           ws0ht[...], bs0[...])
    s, vx, vy, vz = _gvp16(s, vx, vy, vz, wh1t[...], wu1t[...], ws1st[...],
                           ws1ht[...], bs1[...])
    vhx = _dot(vx, wh2t[...])
    vhy = _dot(vy, wh2t[...])
    vhz = _dot(vz, wh2t[...])
    sh = jnp.sqrt(vhx * vhx + vhy * vhy + vhz * vhz + 1e-8)
    so = jnp.maximum(_dot(s, ws2st[...]) + _dot(sh, ws2ht[...]) + bs2[...],
                     0.0)
    s_o[...] = _dot(so, wt[...]) + b[...]
    v_o[:, 0:1] = _dot(vhx, wu2t[...])
    v_o[:, 1:2] = _dot(vhy, wu2t[...])
    v_o[:, 2:3] = _dot(vhz, wu2t[...])


def _noise_weights(noise):
    g0, g1, g2 = noise['gvps']
    return [
        g0['Wh'].T, g0['Wu'].T, g0['Ws'][:, :S].T, g0['Ws'][:, S:].T,
        g0['bs'][None, :],
        g1['Wh'].T, g1['Wu'].T, g1['Ws'][:, :S].T, g1['Ws'][:, S:].T,
        g1['bs'][None, :],
        g2['Wh'].T, g2['Wu'].T, g2['Ws'][:, :S].T, g2['Ws'][:, S:].T,
        g2['bs'][None, :],
        noise['W'].T, noise['b'][None, :],
    ]


def _run_noise(s, v, wts):
    nn = s.shape[0]
    grid = (nn // NB,)
    nspec = lambda w: pl.BlockSpec((NB, w), lambda i: (i, 0))
    wspec = lambda a: pl.BlockSpec(a.shape, lambda i: (0,) * a.ndim)
    return pl.pallas_call(
        _noise_body,
        grid=grid,
        in_specs=[nspec(S), nspec(3 * VD)] + [wspec(a) for a in wts],
        out_specs=[nspec(OUT), nspec(3)],
        out_shape=[jax.ShapeDtypeStruct((nn, OUT), _F32),
                   jax.ShapeDtypeStruct((nn, 3), _F32)],
    )(s, v, *wts)


def kernel(pharm_scalars, pharm_pos, pharm_vectors, prot_scalars, prot_pos,
           prot_vectors, params, edge_index_ff, edge_index_pf, edge_index_fp,
           edge_index_pp, pharm_batch_idx, prot_batch_idx):
    np_, nr = pharm_scalars.shape[0], prot_scalars.shape[0]
    nnodes = {'pharm': np_, 'prot': nr}
    edges = {'ff': edge_index_ff, 'pf': edge_index_pf,
             'fp': edge_index_fp, 'pp': edge_index_pp}

    def planes(v):
        return jnp.transpose(v, (0, 2, 1)).reshape(v.shape[0], 3 * VD)

    pos_pad = {'pharm': jnp.pad(pharm_pos, ((0, 0), (0, 5))),
               'prot': jnp.pad(prot_pos, ((0, 0), (0, 5)))}
    data = {'pharm': (pharm_scalars, planes(pharm_vectors)),
            'prot': (prot_scalars, planes(prot_vectors))}

    for layer in params['convs']:
        sg, vg, ps, pd = [], [], [], []
        for et, st, dt in ETYPES:
            src, dst = edges[et][0], edges[et][1]
            sg.append(jnp.take(data[st][0], src, axis=0))
            vg.append(jnp.take(data[st][1], src, axis=0))
            ps.append(jnp.take(pos_pad[st], src, axis=0))
            pd.append(jnp.take(pos_pad[dt], dst, axis=0))
        sm, vm = _run_msg(jnp.concatenate(sg), jnp.concatenate(vg),
                          jnp.concatenate(ps), jnp.concatenate(pd),
                          _msg_weights(layer))
        dst_ph = jnp.concatenate([edges['ff'][1], edges['pf'][1]])
        dst_pr = jnp.concatenate([edges['fp'][1], edges['pp'][1]])
        ags_ph = jax.ops.segment_sum(sm[:2 * E], dst_ph, num_segments=np_)
        agv_ph = jax.ops.segment_sum(vm[:2 * E], dst_ph, num_segments=np_)
        ags_pr = jax.ops.segment_sum(sm[2 * E:], dst_pr, num_segments=nr)
        agv_pr = jax.ops.segment_sum(vm[2 * E:], dst_pr, num_segments=nr)

        s_new, v_new = _run_upd(
            jnp.concatenate([data['pharm'][0], data['prot'][0]]),
            jnp.concatenate([data['pharm'][1], data['prot'][1]]),
            jnp.concatenate([ags_ph, ags_pr]),
            jnp.concatenate([agv_ph, agv_pr]),
            _upd_weights(layer), np_ // NB)
        data = {'pharm': (s_new[:np_], v_new[:np_]),
                'prot': (s_new[np_:], v_new[np_:])}

    s_out, v_out = _run_noise(data['pharm'][0], data['pharm'][1],
                              _noise_weights(params['noise']))
    return (s_out, v_out)


# speedup vs baseline: 4.6249x; 1.2727x over previous
import functools

import jax
import jax.numpy as jnp
from jax import lax
from jax.experimental import pallas as pl
from jax.experimental.pallas import tpu as pltpu
from jax.experimental.pallas import tpu_sc as plsc

S = 128
VD = 16
E = 160000
MSG_NORM = 10.0
INTER = 64
OUT = 128
ETYPES = [('ff', 'pharm', 'pharm'), ('pf', 'prot', 'pharm'),
          ('fp', 'pharm', 'prot'), ('pp', 'prot', 'prot')]

MW = S + 3 * VD
EB = 1280
NB = 1000
SC_NC = 2
SC_NS = 16
CH = 80

_F32 = jnp.float32


def _dot(a, b):
    return jax.lax.dot_general(a, b, (((1,), (0,)), ((), ())),
                               preferred_element_type=_F32)


def _gvp16(s, vx, vy, vz, wht, wut, wsst, wsht, bs):
    vhx = _dot(vx, wht)
    vhy = _dot(vy, wht)
    vhz = _dot(vz, wht)
    sh = jnp.sqrt(vhx * vhx + vhy * vhy + vhz * vhz + 1e-8)
    so = jnp.maximum(_dot(s, wsst) + _dot(sh, wsht) + bs, 0.0)
    vux = _dot(vhx, wut)
    vuy = _dot(vhy, wut)
    vuz = _dot(vhz, wut)
    n = jnp.sqrt(vux * vux + vuy * vuy + vuz * vuz + 1e-8)
    g = jax.nn.sigmoid(n)
    return so, vux * g, vuy * g, vuz * g


def _msg_body(sg, vg, ps, pd,
              a0t, w0l, wu0t, ws0st, ws0d, ws0ht, bs0,
              wh1t, wu1t, ws1st, ws1ht, bs1,
              wh2t, wu2t, ws2st, ws2ht, bs2,
              ms_o, mv_o):
    s = sg[...]
    vx = vg[:, 0:VD]
    vy = vg[:, VD:2 * VD]
    vz = vg[:, 2 * VD:3 * VD]
    rx = ps[:, 0:1] - pd[:, 0:1]
    ry = ps[:, 1:2] - pd[:, 1:2]
    rz = ps[:, 2:3] - pd[:, 2:3]
    dist = jnp.sqrt(rx * rx + ry * ry + rz * rz + 1e-8)

    vhx = _dot(vx, a0t[0]) + rx * w0l[0]
    vhy = _dot(vy, a0t[0]) + ry * w0l[0]
    vhz = _dot(vz, a0t[0]) + rz * w0l[0]
    sh = jnp.sqrt(vhx * vhx + vhy * vhy + vhz * vhz + 1e-8)
    so = jnp.maximum(_dot(s, ws0st[0]) + dist * ws0d[0] + _dot(sh, ws0ht[0])
                     + bs0[0], 0.0)
    vux = _dot(vhx, wu0t[0])
    vuy = _dot(vhy, wu0t[0])
    vuz = _dot(vhz, wu0t[0])
    n = jnp.sqrt(vux * vux + vuy * vuy + vuz * vuz + 1e-8)
    g = jax.nn.sigmoid(n)
    s, vx, vy, vz = so, vux * g, vuy * g, vuz * g

    s, vx, vy, vz = _gvp16(s, vx, vy, vz, wh1t[0], wu1t[0], ws1st[0],
                           ws1ht[0], bs1[0])
    s, vx, vy, vz = _gvp16(s, vx, vy, vz, wh2t[0], wu2t[0], ws2st[0],
                           ws2ht[0], bs2[0])

    inv = _F32(1.0 / MSG_NORM)
    ms_o[...] = s * inv
    mv_o[:, 0:VD] = vx * inv
    mv_o[:, VD:2 * VD] = vy * inv
    mv_o[:, 2 * VD:3 * VD] = vz * inv
    mv_o[:, 3 * VD:S] = jnp.zeros((s.shape[0], S - 3 * VD), _F32)


def _msg_weights(layer):
    per_type = []
    for et, _st, _dt in ETYPES:
        g0, g1, g2 = layer['msg'][et]
        w = [
            g0['Wh'][:, :VD].T,
            g0['Wh'][:, VD:].T,
            g0['Wu'].T,
            g0['Ws'][:, :S].T,
            g0['Ws'][:, S:S + 1].T,
            g0['Ws'][:, S + 1:].T,
            g0['bs'][None, :],
        ]
        for g in (g1, g2):
            w += [
                g['Wh'].T,
                g['Wu'].T,
                g['Ws'][:, :S].T,
                g['Ws'][:, S:].T,
                g['bs'][None, :],
            ]
        per_type.append(w)
    return [jnp.stack([pt[k] for pt in per_type])
            for k in range(len(per_type[0]))]


def _run_msg(sg, vg, ps, pd, wts):
    ne = sg.shape[0]
    grid = (ne // EB,)
    bpt = E // EB
    espec = lambda w: pl.BlockSpec((EB, w), lambda i: (i, 0))
    wspec = lambda a: pl.BlockSpec((1,) + a.shape[1:],
                                   lambda i: (i // bpt, 0, 0))
    return pl.pallas_call(
        _msg_body,
        grid=grid,
        in_specs=[espec(S), espec(3 * VD), espec(8), espec(8)]
                 + [wspec(a) for a in wts],
        out_specs=[espec(S), espec(S)],
        out_shape=[jax.ShapeDtypeStruct((ne, S), _F32),
                   jax.ShapeDtypeStruct((ne, S), _F32)],
    )(sg, vg, ps, pd, *wts)


def _scatter_body(msgs, idxm, zeros, out, idxr, rows_v, acc, sem, isem):
    c = lax.axis_index("c")
    s = lax.axis_index("s")
    w = c * SC_NS + s
    pw = msgs.shape[0] // (SC_NC * SC_NS)
    nch = pw // CH
    @pl.when(s == 0)
    def _():
        pltpu.sync_copy(zeros, acc)
    plsc.subcore_barrier()
    base = w * pw
    row0 = w * nch
    pltpu.async_copy(msgs.at[pl.ds(base, CH)], rows_v.at[0], sem)
    pltpu.async_copy(idxm.at[0 + row0], idxr.at[0], isem)
    def chunk(j, carry):
        b = lax.rem(j, 2)
        @pl.when(j + 1 < nch)
        def _():
            nb = lax.rem(j + 1, 2)
            pltpu.async_copy(msgs.at[pl.ds(base + (j + 1) * CH, CH)],
                             rows_v.at[nb], sem)
            pltpu.async_copy(idxm.at[j + 1 + row0], idxr.at[nb], isem)
        pltpu.make_async_copy(msgs.at[pl.ds(base + j * CH, CH)],
                              rows_v.at[b], sem).wait()
        pltpu.make_async_copy(idxm.at[j + row0], idxr.at[b], isem).wait()
        pltpu.sync_copy(rows_v.at[b], acc.at[idxr.at[b, 0]], add=True)
        return carry
    lax.fori_loop(0, nch, chunk, 0)
    plsc.subcore_barrier()
    @pl.when(s == 0)
    def _():
        pltpu.sync_copy(acc, out.at[c])


def _run_scatter(msgs, idx, zeros):
    ne, nn = msgs.shape[0], zeros.shape[0]
    idxm = idx.reshape(ne // CH, 1, CH)
    kfn = pl.kernel(
        _scatter_body,
        mesh=plsc.VectorSubcoreMesh(core_axis_name="c", subcore_axis_name="s"),
        out_type=jax.ShapeDtypeStruct((SC_NC, nn, S), _F32),
        scratch_types=[
            pltpu.VMEM((2, 1, CH), jnp.int32),
            pltpu.VMEM((2, CH, S), _F32),
            pltpu.VMEM_SHARED((nn, S), _F32),
            pltpu.SemaphoreType.DMA,
            pltpu.SemaphoreType.DMA,
        ],
    )
    return kfn(msgs, idxm, zeros)


def _upd_body(s0, v0, p0s, p1s, p0v, p1v,
              a0t, b0t, wu0t, wss0t, wsa0t, wsh0t, bs0,
              wh1t, wu1t, ws1st, ws1ht, bs1,
              s_o, v_o):
    vx0 = v0[:, 0:VD]
    vy0 = v0[:, VD:2 * VD]
    vz0 = v0[:, 2 * VD:3 * VD]
    ags = p0s[...] + p1s[...]
    agv = p0v[...] + p1v[...]
    ax = agv[:, 0:VD]
    ay = agv[:, VD:2 * VD]
    az = agv[:, 2 * VD:3 * VD]
    vhx = _dot(vx0, a0t[0]) + _dot(ax, b0t[0])
    vhy = _dot(vy0, a0t[0]) + _dot(ay, b0t[0])
    vhz = _dot(vz0, a0t[0]) + _dot(az, b0t[0])
    sh = jnp.sqrt(vhx * vhx + vhy * vhy + vhz * vhz + 1e-8)
    so = jnp.maximum(_dot(s0[...], wss0t[0]) + _dot(ags, wsa0t[0])
                     + _dot(sh, wsh0t[0]) + bs0[0], 0.0)
    vux = _dot(vhx, wu0t[0])
    vuy = _dot(vhy, wu0t[0])
    vuz = _dot(vhz, wu0t[0])
    n = jnp.sqrt(vux * vux + vuy * vuy + vuz * vuz + 1e-8)
    g = jax.nn.sigmoid(n)
    s, vx, vy, vz = so, vux * g, vuy * g, vuz * g

    s, vx, vy, vz = _gvp16(s, vx, vy, vz, wh1t[0], wu1t[0], ws1st[0],
                           ws1ht[0], bs1[0])
    s_o[...] = s0[...] + s
    v_o[:, 0:VD] = vx0 + vx
    v_o[:, VD:2 * VD] = vy0 + vy
    v_o[:, 2 * VD:3 * VD] = vz0 + vz


def _upd_weights(layer):
    per_type = []
    for t in ('pharm', 'prot'):
        g0, g1 = layer['upd'][t]
        w = [
            g0['Wh'][:, :VD].T,
            g0['Wh'][:, VD:].T,
            g0['Wu'].T,
            g0['Ws'][:, :S].T,
            g0['Ws'][:, S:2 * S].T,
            g0['Ws'][:, 2 * S:].T,
            g0['bs'][None, :],
            g1['Wh'].T,
            g1['Wu'].T,
            g1['Ws'][:, :S].T,
            g1['Ws'][:, S:].T,
            g1['bs'][None, :],
        ]
        per_type.append(w)
    return [jnp.stack([pt[k] for pt in per_type])
            for k in range(len(per_type[0]))]


def _run_upd(s0, v0, p0s, p1s, p0v, p1v, wts, bpt):
    nn = s0.shape[0]
    grid = (nn // NB,)
    nspec = lambda w: pl.BlockSpec((NB, w), lambda i: (i, 0))
    wspec = lambda a: pl.BlockSpec((1,) + a.shape[1:],
                                   lambda i: (i // bpt, 0, 0))
    return pl.pallas_call(
        _upd_body,
        grid=grid,
        in_specs=[nspec(S), nspec(3 * VD), nspec(S), nspec(S), nspec(S),
                  nspec(S)] + [wspec(a) for a in wts],
        out_specs=[nspec(S), nspec(3 * VD)],
        out_shape=[jax.ShapeDtypeStruct((nn, S), _F32),
                   jax.ShapeDtypeStruct((nn, 3 * VD), _F32)],
    )(s0, v0, p0s, p1s, p0v, p1v, *wts)


def _noise_body(s_in, v_in,
                wh0t, wu0t, ws0st, ws0ht, bs0,
                wh1t, wu1t, ws1st, ws1ht, bs1,
                wh2t, wu2t, ws2st, ws2ht, bs2,
                wt, b,
                s_o, v_o):
    s = s_in[...]
    vx = v_in[:, 0:VD]
    vy = v_in[:, VD:2 * VD]
    vz = v_in[:, 2 * VD:3 * VD]
    s, vx, vy, vz = _gvp16(s, vx, vy, vz, wh0t[...], wu0t[...], ws0st[...],
                           ws0ht[...], bs0[...])
    s, vx, vy, vz = _gvp16(s, vx, vy, vz, wh1t[...], wu1t[...], ws1st[...],
                           ws1ht[...], bs1[...])
    vhx = _dot(vx, wh2t[...])
    vhy = _dot(vy, wh2t[...])
    vhz = _dot(vz, wh2t[...])
    sh = jnp.sqrt(vhx * vhx + vhy * vhy + vhz * vhz + 1e-8)
    so = jnp.maximum(_dot(s, ws2st[...]) + _dot(sh, ws2ht[...]) + bs2[...],
                     0.0)
    s_o[...] = _dot(so, wt[...]) + b[...]
    v_o[:, 0:1] = _dot(vhx, wu2t[...])
    v_o[:, 1:2] = _dot(vhy, wu2t[...])
    v_o[:, 2:3] = _dot(vhz, wu2t[...])


def _noise_weights(noise):
    g0, g1, g2 = noise['gvps']
    return [
        g0['Wh'].T, g0['Wu'].T, g0['Ws'][:, :S].T, g0['Ws'][:, S:].T,
        g0['bs'][None, :],
        g1['Wh'].T, g1['Wu'].T, g1['Ws'][:, :S].T, g1['Ws'][:, S:].T,
        g1['bs'][None, :],
        g2['Wh'].T, g2['Wu'].T, g2['Ws'][:, :S].T, g2['Ws'][:, S:].T,
        g2['bs'][None, :],
        noise['W'].T, noise['b'][None, :],
    ]


def _run_noise(s, v, wts):
    nn = s.shape[0]
    grid = (nn // NB,)
    nspec = lambda w: pl.BlockSpec((NB, w), lambda i: (i, 0))
    wspec = lambda a: pl.BlockSpec(a.shape, lambda i: (0,) * a.ndim)
    return pl.pallas_call(
        _noise_body,
        grid=grid,
        in_specs=[nspec(S), nspec(3 * VD)] + [wspec(a) for a in wts],
        out_specs=[nspec(OUT), nspec(3)],
        out_shape=[jax.ShapeDtypeStruct((nn, OUT), _F32),
                   jax.ShapeDtypeStruct((nn, 3), _F32)],
    )(s, v, *wts)


def kernel(pharm_scalars, pharm_pos, pharm_vectors, prot_scalars, prot_pos,
           prot_vectors, params, edge_index_ff, edge_index_pf, edge_index_fp,
           edge_index_pp, pharm_batch_idx, prot_batch_idx):
    np_, nr = pharm_scalars.shape[0], prot_scalars.shape[0]
    nnodes = {'pharm': np_, 'prot': nr}
    edges = {'ff': edge_index_ff, 'pf': edge_index_pf,
             'fp': edge_index_fp, 'pp': edge_index_pp}

    def planes(v):
        return jnp.transpose(v, (0, 2, 1)).reshape(v.shape[0], 3 * VD)

    pos_pad = {'pharm': jnp.pad(pharm_pos, ((0, 0), (0, 5))),
               'prot': jnp.pad(prot_pos, ((0, 0), (0, 5)))}
    data = {'pharm': (pharm_scalars, planes(pharm_vectors)),
            'prot': (prot_scalars, planes(prot_vectors))}

    zeros_nn = jnp.zeros((np_, S), _F32)
    dst_ph = jnp.concatenate([edges['ff'][1], edges['pf'][1]])
    dst_pr = jnp.concatenate([edges['fp'][1], edges['pp'][1]])

    for layer in params['convs']:
        sg, vg, ps, pd = [], [], [], []
        for et, st, dt in ETYPES:
            src, dst = edges[et][0], edges[et][1]
            sg.append(jnp.take(data[st][0], src, axis=0))
            vg.append(jnp.take(data[st][1], src, axis=0))
            ps.append(jnp.take(pos_pad[st], src, axis=0))
            pd.append(jnp.take(pos_pad[dt], dst, axis=0))
        msv = _run_msg(jnp.concatenate(sg), jnp.concatenate(vg),
                       jnp.concatenate(ps), jnp.concatenate(pd),
                       _msg_weights(layer))
        ms, mv = msv
        acc_s_ph = _run_scatter(ms[:2 * E], dst_ph, zeros_nn)
        acc_v_ph = _run_scatter(mv[:2 * E], dst_ph, zeros_nn)
        acc_s_pr = _run_scatter(ms[2 * E:], dst_pr, zeros_nn)
        acc_v_pr = _run_scatter(mv[2 * E:], dst_pr, zeros_nn)
        p0s = jnp.concatenate([acc_s_ph[0], acc_s_pr[0]])
        p1s = jnp.concatenate([acc_s_ph[1], acc_s_pr[1]])
        p0v = jnp.concatenate([acc_v_ph[0], acc_v_pr[0]])
        p1v = jnp.concatenate([acc_v_ph[1], acc_v_pr[1]])

        s_new, v_new = _run_upd(
            jnp.concatenate([data['pharm'][0], data['prot'][0]]),
            jnp.concatenate([data['pharm'][1], data['prot'][1]]),
            p0s, p1s, p0v, p1v, _upd_weights(layer), np_ // NB)
        data = {'pharm': (s_new[:np_], v_new[:np_]),
                'prot': (s_new[np_:], v_new[np_:])}

    s_out, v_out = _run_noise(data['pharm'][0], data['pharm'][1],
                              _noise_weights(params['noise']))
    return (s_out, v_out)


# speedup vs baseline: 4.6480x; 1.0050x over previous
import functools

import jax
import jax.numpy as jnp
from jax import lax
from jax.experimental import pallas as pl
from jax.experimental.pallas import tpu as pltpu
from jax.experimental.pallas import tpu_sc as plsc

S = 128
VD = 16
E = 160000
MSG_NORM = 10.0
INTER = 64
OUT = 128
ETYPES = [('ff', 'pharm', 'pharm'), ('pf', 'prot', 'pharm'),
          ('fp', 'pharm', 'prot'), ('pp', 'prot', 'prot')]

MW = S + 3 * VD
EB = 1280
NB = 1000
SC_NC = 2
SC_NS = 16
CH = 80

_F32 = jnp.float32


def _dot(a, b):
    return jax.lax.dot_general(a, b, (((1,), (0,)), ((), ())),
                               preferred_element_type=_F32)


def _gvp16(s, vx, vy, vz, wht, wut, wsst, wsht, bs):
    vhx = _dot(vx, wht)
    vhy = _dot(vy, wht)
    vhz = _dot(vz, wht)
    sh = jnp.sqrt(vhx * vhx + vhy * vhy + vhz * vhz + 1e-8)
    so = jnp.maximum(_dot(s, wsst) + _dot(sh, wsht) + bs, 0.0)
    vux = _dot(vhx, wut)
    vuy = _dot(vhy, wut)
    vuz = _dot(vhz, wut)
    n = jnp.sqrt(vux * vux + vuy * vuy + vuz * vuz + 1e-8)
    g = jax.nn.sigmoid(n)
    return so, vux * g, vuy * g, vuz * g


def _msg_body(sg, vg, ps, pd,
              a0t, w0l, wu0t, ws0st, ws0d, ws0ht, bs0,
              wh1t, wu1t, ws1st, ws1ht, bs1,
              wh2t, wu2t, ws2st, ws2ht, bs2,
              ms_o, mv_o):
    s = sg[...]
    vx = vg[:, 0:VD]
    vy = vg[:, VD:2 * VD]
    vz = vg[:, 2 * VD:3 * VD]
    rx = ps[:, 0:1] - pd[:, 0:1]
    ry = ps[:, 1:2] - pd[:, 1:2]
    rz = ps[:, 2:3] - pd[:, 2:3]
    dist = jnp.sqrt(rx * rx + ry * ry + rz * rz + 1e-8)

    vhx = _dot(vx, a0t[0]) + rx * w0l[0]
    vhy = _dot(vy, a0t[0]) + ry * w0l[0]
    vhz = _dot(vz, a0t[0]) + rz * w0l[0]
    sh = jnp.sqrt(vhx * vhx + vhy * vhy + vhz * vhz + 1e-8)
    so = jnp.maximum(_dot(s, ws0st[0]) + dist * ws0d[0] + _dot(sh, ws0ht[0])
                     + bs0[0], 0.0)
    vux = _dot(vhx, wu0t[0])
    vuy = _dot(vhy, wu0t[0])
    vuz = _dot(vhz, wu0t[0])
    n = jnp.sqrt(vux * vux + vuy * vuy + vuz * vuz + 1e-8)
    g = jax.nn.sigmoid(n)
    s, vx, vy, vz = so, vux * g, vuy * g, vuz * g

    s, vx, vy, vz = _gvp16(s, vx, vy, vz, wh1t[0], wu1t[0], ws1st[0],
                           ws1ht[0], bs1[0])
    s, vx, vy, vz = _gvp16(s, vx, vy, vz, wh2t[0], wu2t[0], ws2st[0],
                           ws2ht[0], bs2[0])

    inv = _F32(1.0 / MSG_NORM)
    ms_o[...] = s * inv
    mv_o[:, 0:VD] = vx * inv
    mv_o[:, VD:2 * VD] = vy * inv
    mv_o[:, 2 * VD:3 * VD] = vz * inv
    mv_o[:, 3 * VD:S] = jnp.zeros((s.shape[0], S - 3 * VD), _F32)


def _msg_weights(layer):
    per_type = []
    for et, _st, _dt in ETYPES:
        g0, g1, g2 = layer['msg'][et]
        w = [
            g0['Wh'][:, :VD].T,
            g0['Wh'][:, VD:].T,
            g0['Wu'].T,
            g0['Ws'][:, :S].T,
            g0['Ws'][:, S:S + 1].T,
            g0['Ws'][:, S + 1:].T,
            g0['bs'][None, :],
        ]
        for g in (g1, g2):
            w += [
                g['Wh'].T,
                g['Wu'].T,
                g['Ws'][:, :S].T,
                g['Ws'][:, S:].T,
                g['bs'][None, :],
            ]
        per_type.append(w)
    return [jnp.stack([pt[k] for pt in per_type])
            for k in range(len(per_type[0]))]


def _run_msg(sg, vg, ps, pd, wts):
    ne = sg.shape[0]
    grid = (ne // EB,)
    bpt = E // EB
    espec = lambda w: pl.BlockSpec((EB, w), lambda i: (i, 0))
    wspec = lambda a: pl.BlockSpec((1,) + a.shape[1:],
                                   lambda i: (i // bpt, 0, 0))
    return pl.pallas_call(
        _msg_body,
        grid=grid,
        in_specs=[espec(S), espec(3 * VD), espec(8), espec(8)]
                 + [wspec(a) for a in wts],
        out_specs=[espec(S), espec(S)],
        out_shape=[jax.ShapeDtypeStruct((ne, S), _F32),
                   jax.ShapeDtypeStruct((ne, S), _F32)],
    )(sg, vg, ps, pd, *wts)


def _scatter_body(msgs, idxm, zeros, out, idxr, rows_v, acc, sem, isem, ssem):
    c = lax.axis_index("c")
    s = lax.axis_index("s")
    w = c * SC_NS + s
    pw = msgs.shape[0] // (SC_NC * SC_NS)
    nch = pw // CH
    @pl.when(s == 0)
    def _():
        pltpu.sync_copy(zeros, acc)
    plsc.subcore_barrier()
    base = w * pw
    row0 = w * nch

    def load(j, buf):
        pltpu.async_copy(msgs.at[pl.ds(base + j * CH, CH)],
                         rows_v.at[buf], sem)
        pltpu.async_copy(idxm.at[row0 + j], idxr.at[buf], isem)

    load(0, 0)
    load(1, 1)
    def chunk(j, carry):
        b = lax.rem(j, 4)
        pltpu.make_async_copy(msgs.at[pl.ds(base + j * CH, CH)],
                              rows_v.at[b], sem).wait()
        pltpu.make_async_copy(idxm.at[row0 + j], idxr.at[b], isem).wait()
        pltpu.async_copy(rows_v.at[b], acc.at[idxr.at[b, 0]], ssem, add=True)
        @pl.when(j >= 2)
        def _():
            ob = lax.rem(j + 2, 4)
            pltpu.make_async_copy(rows_v.at[ob], acc.at[idxr.at[ob, 0]],
                                  ssem).wait()
        @pl.when(j + 2 < nch)
        def _():
            load(j + 2, lax.rem(j + 2, 4))
        return carry
    lax.fori_loop(0, nch, chunk, 0)
    pltpu.make_async_copy(rows_v.at[lax.rem(nch - 2, 4)],
                          acc.at[idxr.at[lax.rem(nch - 2, 4), 0]],
                          ssem).wait()
    pltpu.make_async_copy(rows_v.at[lax.rem(nch - 1, 4)],
                          acc.at[idxr.at[lax.rem(nch - 1, 4), 0]],
                          ssem).wait()
    plsc.subcore_barrier()
    @pl.when(s == 0)
    def _():
        pltpu.sync_copy(acc, out.at[c])


def _run_scatter(msgs, idx, zeros):
    ne, nn = msgs.shape[0], zeros.shape[0]
    idxm = idx.reshape(ne // CH, 1, CH)
    kfn = pl.kernel(
        _scatter_body,
        mesh=plsc.VectorSubcoreMesh(core_axis_name="c", subcore_axis_name="s"),
        out_type=jax.ShapeDtypeStruct((SC_NC, nn, S), _F32),
        scratch_types=[
            pltpu.VMEM((4, 1, CH), jnp.int32),
            pltpu.VMEM((4, CH, S), _F32),
            pltpu.VMEM_SHARED((nn, S), _F32),
            pltpu.SemaphoreType.DMA,
            pltpu.SemaphoreType.DMA,
            pltpu.SemaphoreType.DMA,
        ],
    )
    return kfn(msgs, idxm, zeros)


def _upd_body(s0, v0, p0s, p1s, p0v, p1v,
              a0t, b0t, wu0t, wss0t, wsa0t, wsh0t, bs0,
              wh1t, wu1t, ws1st, ws1ht, bs1,
              s_o, v_o):
    vx0 = v0[:, 0:VD]
    vy0 = v0[:, VD:2 * VD]
    vz0 = v0[:, 2 * VD:3 * VD]
    ags = p0s[...] + p1s[...]
    agv = p0v[...] + p1v[...]
    ax = agv[:, 0:VD]
    ay = agv[:, VD:2 * VD]
    az = agv[:, 2 * VD:3 * VD]
    vhx = _dot(vx0, a0t[0]) + _dot(ax, b0t[0])
    vhy = _dot(vy0, a0t[0]) + _dot(ay, b0t[0])
    vhz = _dot(vz0, a0t[0]) + _dot(az, b0t[0])
    sh = jnp.sqrt(vhx * vhx + vhy * vhy + vhz * vhz + 1e-8)
    so = jnp.maximum(_dot(s0[...], wss0t[0]) + _dot(ags, wsa0t[0])
                     + _dot(sh, wsh0t[0]) + bs0[0], 0.0)
    vux = _dot(vhx, wu0t[0])
    vuy = _dot(vhy, wu0t[0])
    vuz = _dot(vhz, wu0t[0])
    n = jnp.sqrt(vux * vux + vuy * vuy + vuz * vuz + 1e-8)
    g = jax.nn.sigmoid(n)
    s, vx, vy, vz = so, vux * g, vuy * g, vuz * g

    s, vx, vy, vz = _gvp16(s, vx, vy, vz, wh1t[0], wu1t[0], ws1st[0],
                           ws1ht[0], bs1[0])
    s_o[...] = s0[...] + s
    v_o[:, 0:VD] = vx0 + vx
    v_o[:, VD:2 * VD] = vy0 + vy
    v_o[:, 2 * VD:3 * VD] = vz0 + vz


def _upd_weights(layer):
    per_type = []
    for t in ('pharm', 'prot'):
        g0, g1 = layer['upd'][t]
        w = [
            g0['Wh'][:, :VD].T,
            g0['Wh'][:, VD:].T,
            g0['Wu'].T,
            g0['Ws'][:, :S].T,
            g0['Ws'][:, S:2 * S].T,
            g0['Ws'][:, 2 * S:].T,
            g0['bs'][None, :],
            g1['Wh'].T,
            g1['Wu'].T,
            g1['Ws'][:, :S].T,
            g1['Ws'][:, S:].T,
            g1['bs'][None, :],
        ]
        per_type.append(w)
    return [jnp.stack([pt[k] for pt in per_type])
            for k in range(len(per_type[0]))]


def _run_upd(s0, v0, p0s, p1s, p0v, p1v, wts, bpt):
    nn = s0.shape[0]
    grid = (nn // NB,)
    nspec = lambda w: pl.BlockSpec((NB, w), lambda i: (i, 0))
    wspec = lambda a: pl.BlockSpec((1,) + a.shape[1:],
                                   lambda i: (i // bpt, 0, 0))
    return pl.pallas_call(
        _upd_body,
        grid=grid,
        in_specs=[nspec(S), nspec(3 * VD), nspec(S), nspec(S), nspec(S),
                  nspec(S)] + [wspec(a) for a in wts],
        out_specs=[nspec(S), nspec(3 * VD)],
        out_shape=[jax.ShapeDtypeStruct((nn, S), _F32),
                   jax.ShapeDtypeStruct((nn, 3 * VD), _F32)],
    )(s0, v0, p0s, p1s, p0v, p1v, *wts)


def _noise_body(s_in, v_in,
                wh0t, wu0t, ws0st, ws0ht, bs0,
                wh1t, wu1t, ws1st, ws1ht, bs1,
                wh2t, wu2t, ws2st, ws2ht, bs2,
                wt, b,
                s_o, v_o):
    s = s_in[...]
    vx = v_in[:, 0:VD]
    vy = v_in[:, VD:2 * VD]
    vz = v_in[:, 2 * VD:3 * VD]
    s, vx, vy, vz = _gvp16(s, vx, vy, vz, wh0t[...], wu0t[...], ws0st[...],
                           ws0ht[...], bs0[...])
    s, vx, vy, vz = _gvp16(s, vx, vy, vz, wh1t[...], wu1t[...], ws1st[...],
                           ws1ht[...], bs1[...])
    vhx = _dot(vx, wh2t[...])
    vhy = _dot(vy, wh2t[...])
    vhz = _dot(vz, wh2t[...])
    sh = jnp.sqrt(vhx * vhx + vhy * vhy + vhz * vhz + 1e-8)
    so = jnp.maximum(_dot(s, ws2st[...]) + _dot(sh, ws2ht[...]) + bs2[...],
                     0.0)
    s_o[...] = _dot(so, wt[...]) + b[...]
    v_o[:, 0:1] = _dot(vhx, wu2t[...])
    v_o[:, 1:2] = _dot(vhy, wu2t[...])
    v_o[:, 2:3] = _dot(vhz, wu2t[...])


def _noise_weights(noise):
    g0, g1, g2 = noise['gvps']
    return [
        g0['Wh'].T, g0['Wu'].T, g0['Ws'][:, :S].T, g0['Ws'][:, S:].T,
        g0['bs'][None, :],
        g1['Wh'].T, g1['Wu'].T, g1['Ws'][:, :S].T, g1['Ws'][:, S:].T,
        g1['bs'][None, :],
        g2['Wh'].T, g2['Wu'].T, g2['Ws'][:, :S].T, g2['Ws'][:, S:].T,
        g2['bs'][None, :],
        noise['W'].T, noise['b'][None, :],
    ]


def _run_noise(s, v, wts):
    nn = s.shape[0]
    grid = (nn // NB,)
    nspec = lambda w: pl.BlockSpec((NB, w), lambda i: (i, 0))
    wspec = lambda a: pl.BlockSpec(a.shape, lambda i: (0,) * a.ndim)
    return pl.pallas_call(
        _noise_body,
        grid=grid,
        in_specs=[nspec(S), nspec(3 * VD)] + [wspec(a) for a in wts],
        out_specs=[nspec(OUT), nspec(3)],
        out_shape=[jax.ShapeDtypeStruct((nn, OUT), _F32),
                   jax.ShapeDtypeStruct((nn, 3), _F32)],
    )(s, v, *wts)


def kernel(pharm_scalars, pharm_pos, pharm_vectors, prot_scalars, prot_pos,
           prot_vectors, params, edge_index_ff, edge_index_pf, edge_index_fp,
           edge_index_pp, pharm_batch_idx, prot_batch_idx):
    np_, nr = pharm_scalars.shape[0], prot_scalars.shape[0]
    nnodes = {'pharm': np_, 'prot': nr}
    edges = {'ff': edge_index_ff, 'pf': edge_index_pf,
             'fp': edge_index_fp, 'pp': edge_index_pp}

    def planes(v):
        return jnp.transpose(v, (0, 2, 1)).reshape(v.shape[0], 3 * VD)

    pos_pad = {'pharm': jnp.pad(pharm_pos, ((0, 0), (0, 5))),
               'prot': jnp.pad(prot_pos, ((0, 0), (0, 5)))}
    data = {'pharm': (pharm_scalars, planes(pharm_vectors)),
            'prot': (prot_scalars, planes(prot_vectors))}

    zeros_nn = jnp.zeros((np_, S), _F32)
    dst_ph = jnp.concatenate([edges['ff'][1], edges['pf'][1]])
    dst_pr = jnp.concatenate([edges['fp'][1], edges['pp'][1]])
    for layer in params['convs']:
        sg, vg, ps, pd = [], [], [], []
        for et, st, dt in ETYPES:
            src_i, dst_i = edges[et][0], edges[et][1]
            sg.append(jnp.take(data[st][0], src_i, axis=0))
            vg.append(jnp.take(data[st][1], src_i, axis=0))
            ps.append(jnp.take(pos_pad[st], src_i, axis=0))
            pd.append(jnp.take(pos_pad[dt], dst_i, axis=0))
        ms, mv = _run_msg(jnp.concatenate(sg), jnp.concatenate(vg),
                          jnp.concatenate(ps), jnp.concatenate(pd),
                          _msg_weights(layer))
        acc_s_ph = _run_scatter(ms[:2 * E], dst_ph, zeros_nn)
        acc_v_ph = _run_scatter(mv[:2 * E], dst_ph, zeros_nn)
        acc_s_pr = _run_scatter(ms[2 * E:], dst_pr, zeros_nn)
        acc_v_pr = _run_scatter(mv[2 * E:], dst_pr, zeros_nn)
        p0s = jnp.concatenate([acc_s_ph[0], acc_s_pr[0]])
        p1s = jnp.concatenate([acc_s_ph[1], acc_s_pr[1]])
        p0v = jnp.concatenate([acc_v_ph[0], acc_v_pr[0]])
        p1v = jnp.concatenate([acc_v_ph[1], acc_v_pr[1]])

        s_new, v_new = _run_upd(
            jnp.concatenate([data['pharm'][0], data['prot'][0]]),
            jnp.concatenate([data['pharm'][1], data['prot'][1]]),
            p0s, p1s, p0v, p1v, _upd_weights(layer), np_ // NB)
        data = {'pharm': (s_new[:np_], v_new[:np_]),
                'prot': (s_new[np_:], v_new[np_:])}

    s_out, v_out = _run_noise(data['pharm'][0], data['pharm'][1],
                              _noise_weights(params['noise']))
    return (s_out, v_out)


# speedup vs baseline: 4.6814x; 1.0072x over previous
import jax
import jax.numpy as jnp
from jax import lax
from jax.experimental import pallas as pl
from jax.experimental.pallas import tpu as pltpu
from jax.experimental.pallas import tpu_sc as plsc

S = 128
VD = 16
E = 160000
MSG_NORM = 10.0
INTER = 64
OUT = 128
ETYPES = [('ff', 'pharm', 'pharm'), ('pf', 'prot', 'pharm'),
          ('fp', 'pharm', 'prot'), ('pp', 'prot', 'prot')]

EB = 1280
NB = 1000
SC_NC = 2
SC_NS = 16
CH = 80

_F32 = jnp.float32


def _dot(a, b):
    return jax.lax.dot_general(a, b, (((1,), (0,)), ((), ())),
                               preferred_element_type=_F32)


def _gvp16(s, vx, vy, vz, wht, wut, wst, bs):
    vhx = _dot(vx, wht)
    vhy = _dot(vy, wht)
    vhz = _dot(vz, wht)
    sh = jnp.sqrt(vhx * vhx + vhy * vhy + vhz * vhz + 1e-8)
    so = jnp.maximum(_dot(jnp.concatenate([s, sh], axis=1), wst) + bs, 0.0)
    vux = _dot(vhx, wut)
    vuy = _dot(vhy, wut)
    vuz = _dot(vhz, wut)
    n = jnp.sqrt(vux * vux + vuy * vuy + vuz * vuz + 1e-8)
    g = jax.nn.sigmoid(n)
    return so, vux * g, vuy * g, vuz * g


def _msg_body(sg, vg, ps, pd,
              wh0t, wu0t, ws0t, bs0,
              wh1t, wu1t, ws1t, bs1,
              wh2t, wu2t, ws2t, bs2,
              ms_o, mv_o):
    s = sg[...]
    vx = vg[:, 0:VD]
    vy = vg[:, VD:2 * VD]
    vz = vg[:, 2 * VD:3 * VD]
    rx = ps[:, 0:1] - pd[:, 0:1]
    ry = ps[:, 1:2] - pd[:, 1:2]
    rz = ps[:, 2:3] - pd[:, 2:3]
    dist = jnp.sqrt(rx * rx + ry * ry + rz * rz + 1e-8)

    vhx = _dot(jnp.concatenate([vx, rx], axis=1), wh0t[0])
    vhy = _dot(jnp.concatenate([vy, ry], axis=1), wh0t[0])
    vhz = _dot(jnp.concatenate([vz, rz], axis=1), wh0t[0])
    sh = jnp.sqrt(vhx * vhx + vhy * vhy + vhz * vhz + 1e-8)
    so = jnp.maximum(
        _dot(jnp.concatenate([s, dist, sh], axis=1), ws0t[0]) + bs0[0], 0.0)
    vux = _dot(vhx, wu0t[0])
    vuy = _dot(vhy, wu0t[0])
    vuz = _dot(vhz, wu0t[0])
    n = jnp.sqrt(vux * vux + vuy * vuy + vuz * vuz + 1e-8)
    g = jax.nn.sigmoid(n)
    s, vx, vy, vz = so, vux * g, vuy * g, vuz * g

    s, vx, vy, vz = _gvp16(s, vx, vy, vz, wh1t[0], wu1t[0], ws1t[0], bs1[0])
    s, vx, vy, vz = _gvp16(s, vx, vy, vz, wh2t[0], wu2t[0], ws2t[0], bs2[0])

    inv = _F32(1.0 / MSG_NORM)
    ms_o[...] = s * inv
    mv_o[:, 0:VD] = vx * inv
    mv_o[:, VD:2 * VD] = vy * inv
    mv_o[:, 2 * VD:3 * VD] = vz * inv
    mv_o[:, 3 * VD:S] = jnp.zeros((s.shape[0], S - 3 * VD), _F32)


def _msg_weights(layer):
    per_type = []
    for et, _st, _dt in ETYPES:
        g0, g1, g2 = layer['msg'][et]
        w = [g0['Wh'].T, g0['Wu'].T, g0['Ws'].T, g0['bs'][None, :]]
        for g in (g1, g2):
            w += [g['Wh'].T, g['Wu'].T, g['Ws'].T, g['bs'][None, :]]
        per_type.append(w)
    return [jnp.stack([pt[k] for pt in per_type])
            for k in range(len(per_type[0]))]


def _run_msg(sg, vg, ps, pd, wts):
    ne = sg.shape[0]
    grid = (ne // EB,)
    bpt = E // EB
    espec = lambda w: pl.BlockSpec((EB, w), lambda i: (i, 0))
    wspec = lambda a: pl.BlockSpec((1,) + a.shape[1:],
                                   lambda i: (i // bpt, 0, 0))
    return pl.pallas_call(
        _msg_body,
        grid=grid,
        in_specs=[espec(S), espec(3 * VD), espec(8), espec(8)]
                 + [wspec(a) for a in wts],
        out_specs=[espec(S), espec(S)],
        out_shape=[jax.ShapeDtypeStruct((ne, S), _F32),
                   jax.ShapeDtypeStruct((ne, S), _F32)],
    )(sg, vg, ps, pd, *wts)


def _scatter_body(msgs, idxm, zeros, out, idxr, rows_v, acc, sem, isem, ssem):
    c = lax.axis_index("c")
    s = lax.axis_index("s")
    w = c * SC_NS + s
    pw = msgs.shape[0] // (SC_NC * SC_NS)
    nch = pw // CH
    @pl.when(s == 0)
    def _():
        pltpu.sync_copy(zeros, acc)
    plsc.subcore_barrier()
    base = w * pw
    row0 = w * nch

    def load(j, buf):
        pltpu.async_copy(msgs.at[pl.ds(base + j * CH, CH)],
                         rows_v.at[buf], sem)
        pltpu.async_copy(idxm.at[row0 + j], idxr.at[buf], isem)

    load(0, 0)
    load(1, 1)
    def chunk(j, carry):
        b = lax.rem(j, 4)
        pltpu.make_async_copy(msgs.at[pl.ds(base + j * CH, CH)],
                              rows_v.at[b], sem).wait()
        pltpu.make_async_copy(idxm.at[row0 + j], idxr.at[b], isem).wait()
        pltpu.async_copy(rows_v.at[b], acc.at[idxr.at[b, 0]], ssem, add=True)
        @pl.when(j >= 2)
        def _():
            ob = lax.rem(j + 2, 4)
            pltpu.make_async_copy(rows_v.at[ob], acc.at[idxr.at[ob, 0]],
                                  ssem).wait()
        @pl.when(j + 2 < nch)
        def _():
            load(j + 2, lax.rem(j + 2, 4))
        return carry
    lax.fori_loop(0, nch, chunk, 0)
    pltpu.make_async_copy(rows_v.at[lax.rem(nch - 2, 4)],
                          acc.at[idxr.at[lax.rem(nch - 2, 4), 0]],
                          ssem).wait()
    pltpu.make_async_copy(rows_v.at[lax.rem(nch - 1, 4)],
                          acc.at[idxr.at[lax.rem(nch - 1, 4), 0]],
                          ssem).wait()
    plsc.subcore_barrier()
    @pl.when(s == 0)
    def _():
        pltpu.sync_copy(acc, out.at[c])


def _run_scatter(msgs, idx, zeros):
    ne, nn = msgs.shape[0], zeros.shape[0]
    idxm = idx.reshape(ne // CH, 1, CH)
    kfn = pl.kernel(
        _scatter_body,
        mesh=plsc.VectorSubcoreMesh(core_axis_name="c", subcore_axis_name="s"),
        out_type=jax.ShapeDtypeStruct((SC_NC, nn, S), _F32),
        scratch_types=[
            pltpu.VMEM((4, 1, CH), jnp.int32),
            pltpu.VMEM((4, CH, S), _F32),
            pltpu.VMEM_SHARED((nn, S), _F32),
            pltpu.SemaphoreType.DMA,
            pltpu.SemaphoreType.DMA,
            pltpu.SemaphoreType.DMA,
        ],
    )
    return kfn(msgs, idxm, zeros)


def _upd_body(s0, v0, p0s, p1s, p0v, p1v,
              wh0t, wu0t, ws0t, bs0,
              wh1t, wu1t, ws1t, bs1,
              s_o, v_o):
    vx0 = v0[:, 0:VD]
    vy0 = v0[:, VD:2 * VD]
    vz0 = v0[:, 2 * VD:3 * VD]
    ags = p0s[...] + p1s[...]
    agv = p0v[...] + p1v[...]
    ax = agv[:, 0:VD]
    ay = agv[:, VD:2 * VD]
    az = agv[:, 2 * VD:3 * VD]
    vhx = _dot(jnp.concatenate([vx0, ax], axis=1), wh0t[0])
    vhy = _dot(jnp.concatenate([vy0, ay], axis=1), wh0t[0])
    vhz = _dot(jnp.concatenate([vz0, az], axis=1), wh0t[0])
    sh = jnp.sqrt(vhx * vhx + vhy * vhy + vhz * vhz + 1e-8)
    so = jnp.maximum(
        _dot(jnp.concatenate([s0[...], ags, sh], axis=1), ws0t[0]) + bs0[0],
        0.0)
    vux = _dot(vhx, wu0t[0])
    vuy = _dot(vhy, wu0t[0])
    vuz = _dot(vhz, wu0t[0])
    n = jnp.sqrt(vux * vux + vuy * vuy + vuz * vuz + 1e-8)
    g = jax.nn.sigmoid(n)
    s, vx, vy, vz = so, vux * g, vuy * g, vuz * g

    s, vx, vy, vz = _gvp16(s, vx, vy, vz, wh1t[0], wu1t[0], ws1t[0], bs1[0])
    s_o[...] = s0[...] + s
    v_o[:, 0:VD] = vx0 + vx
    v_o[:, VD:2 * VD] = vy0 + vy
    v_o[:, 2 * VD:3 * VD] = vz0 + vz


def _upd_weights(layer):
    per_type = []
    for t in ('pharm', 'prot'):
        g0, g1 = layer['upd'][t]
        w = [g0['Wh'].T, g0['Wu'].T, g0['Ws'].T, g0['bs'][None, :],
             g1['Wh'].T, g1['Wu'].T, g1['Ws'].T, g1['bs'][None, :]]
        per_type.append(w)
    return [jnp.stack([pt[k] for pt in per_type])
            for k in range(len(per_type[0]))]


def _run_upd(s0, v0, p0s, p1s, p0v, p1v, wts, bpt):
    nn = s0.shape[0]
    grid = (nn // NB,)
    nspec = lambda w: pl.BlockSpec((NB, w), lambda i: (i, 0))
    wspec = lambda a: pl.BlockSpec((1,) + a.shape[1:],
                                   lambda i: (i // bpt, 0, 0))
    return pl.pallas_call(
        _upd_body,
        grid=grid,
        in_specs=[nspec(S), nspec(3 * VD), nspec(S), nspec(S), nspec(S),
                  nspec(S)] + [wspec(a) for a in wts],
        out_specs=[nspec(S), nspec(3 * VD)],
        out_shape=[jax.ShapeDtypeStruct((nn, S), _F32),
                   jax.ShapeDtypeStruct((nn, 3 * VD), _F32)],
    )(s0, v0, p0s, p1s, p0v, p1v, *wts)


def _noise_body(s_in, v_in,
                wh0t, wu0t, ws0t, bs0,
                wh1t, wu1t, ws1t, bs1,
                wh2t, wu2t, ws2t, bs2,
                wt, b,
                s_o, v_o):
    s = s_in[...]
    vx = v_in[:, 0:VD]
    vy = v_in[:, VD:2 * VD]
    vz = v_in[:, 2 * VD:3 * VD]
    s, vx, vy, vz = _gvp16(s, vx, vy, vz, wh0t[...], wu0t[...], ws0t[...],
                           bs0[...])
    s, vx, vy, vz = _gvp16(s, vx, vy, vz, wh1t[...], wu1t[...], ws1t[...],
                           bs1[...])
    vhx = _dot(vx, wh2t[...])
    vhy = _dot(vy, wh2t[...])
    vhz = _dot(vz, wh2t[...])
    sh = jnp.sqrt(vhx * vhx + vhy * vhy + vhz * vhz + 1e-8)
    so = jnp.maximum(
        _dot(jnp.concatenate([s, sh], axis=1), ws2t[...]) + bs2[...], 0.0)
    s_o[...] = _dot(so, wt[...]) + b[...]
    v_o[:, 0:1] = _dot(vhx, wu2t[...])
    v_o[:, 1:2] = _dot(vhy, wu2t[...])
    v_o[:, 2:3] = _dot(vhz, wu2t[...])


def _noise_weights(noise):
    g0, g1, g2 = noise['gvps']
    return [
        g0['Wh'].T, g0['Wu'].T, g0['Ws'].T, g0['bs'][None, :],
        g1['Wh'].T, g1['Wu'].T, g1['Ws'].T, g1['bs'][None, :],
        g2['Wh'].T, g2['Wu'].T, g2['Ws'].T, g2['bs'][None, :],
        noise['W'].T, noise['b'][None, :],
    ]


def _run_noise(s, v, wts):
    nn = s.shape[0]
    grid = (nn // NB,)
    nspec = lambda w: pl.BlockSpec((NB, w), lambda i: (i, 0))
    wspec = lambda a: pl.BlockSpec(a.shape, lambda i: (0,) * a.ndim)
    return pl.pallas_call(
        _noise_body,
        grid=grid,
        in_specs=[nspec(S), nspec(3 * VD)] + [wspec(a) for a in wts],
        out_specs=[nspec(OUT), nspec(3)],
        out_shape=[jax.ShapeDtypeStruct((nn, OUT), _F32),
                   jax.ShapeDtypeStruct((nn, 3), _F32)],
    )(s, v, *wts)


def kernel(pharm_scalars, pharm_pos, pharm_vectors, prot_scalars, prot_pos,
           prot_vectors, params, edge_index_ff, edge_index_pf, edge_index_fp,
           edge_index_pp, pharm_batch_idx, prot_batch_idx):
    np_, nr = pharm_scalars.shape[0], prot_scalars.shape[0]
    nnodes = {'pharm': np_, 'prot': nr}
    edges = {'ff': edge_index_ff, 'pf': edge_index_pf,
             'fp': edge_index_fp, 'pp': edge_index_pp}

    def planes(v):
        return jnp.transpose(v, (0, 2, 1)).reshape(v.shape[0], 3 * VD)

    pos_pad = {'pharm': jnp.pad(pharm_pos, ((0, 0), (0, 5))),
               'prot': jnp.pad(prot_pos, ((0, 0), (0, 5)))}
    data = {'pharm': (pharm_scalars, planes(pharm_vectors)),
            'prot': (prot_scalars, planes(prot_vectors))}

    zeros_nn = jnp.zeros((np_, S), _F32)
    dst_ph = jnp.concatenate([edges['ff'][1], edges['pf'][1]])
    dst_pr = jnp.concatenate([edges['fp'][1], edges['pp'][1]])
    for layer in params['convs']:
        sg, vg, ps, pd = [], [], [], []
        for et, st, dt in ETYPES:
            src_i, dst_i = edges[et][0], edges[et][1]
            sg.append(jnp.take(data[st][0], src_i, axis=0))
            vg.append(jnp.take(data[st][1], src_i, axis=0))
            ps.append(jnp.take(pos_pad[st], src_i, axis=0))
            pd.append(jnp.take(pos_pad[dt], dst_i, axis=0))
        ms, mv = _run_msg(jnp.concatenate(sg), jnp.concatenate(vg),
                          jnp.concatenate(ps), jnp.concatenate(pd),
                          _msg_weights(layer))
        acc_s_ph = _run_scatter(ms[:2 * E], dst_ph, zeros_nn)
        acc_v_ph = _run_scatter(mv[:2 * E], dst_ph, zeros_nn)
        acc_s_pr = _run_scatter(ms[2 * E:], dst_pr, zeros_nn)
        acc_v_pr = _run_scatter(mv[2 * E:], dst_pr, zeros_nn)
        p0s = jnp.concatenate([acc_s_ph[0], acc_s_pr[0]])
        p1s = jnp.concatenate([acc_s_ph[1], acc_s_pr[1]])
        p0v = jnp.concatenate([acc_v_ph[0], acc_v_pr[0]])
        p1v = jnp.concatenate([acc_v_ph[1], acc_v_pr[1]])

        s_new, v_new = _run_upd(
            jnp.concatenate([data['pharm'][0], data['prot'][0]]),
            jnp.concatenate([data['pharm'][1], data['prot'][1]]),
            p0s, p1s, p0v, p1v, _upd_weights(layer), np_ // NB)
        data = {'pharm': (s_new[:np_], v_new[:np_]),
                'prot': (s_new[np_:], v_new[np_:])}

    s_out, v_out = _run_noise(data['pharm'][0], data['pharm'][1],
                              _noise_weights(params['noise']))
    return (s_out, v_out)
